# Initial kernel scaffold; baseline (speedup 1.0000x reference)
import functools

import jax
import jax.numpy as jnp
from jax import lax
from jax.experimental import pallas as pl
from jax.experimental.pallas import tpu as pltpu
from jax.experimental.pallas import tpu_sc as plsc

N = 10000
E = 320000
D = 128

NC = 2
NS = 16
LANES = 16
C = 80
EPT = E // (NC * NS)
NCHUNK = EPT // C
ROW_BLK = 1000


def _mm_body(x_ref, wk, bk, wq, bq, wv, bv, ws, bs, k_out, q_out, v_out, b_out):
    xb = x_ref[...]
    k_out[...] = jnp.dot(xb, wk[...], preferred_element_type=jnp.float32) + bk[...]
    q_out[...] = jnp.dot(xb, wq[...], preferred_element_type=jnp.float32) + bq[...]
    v_out[...] = jnp.dot(xb, wv[...], preferred_element_type=jnp.float32) + bv[...]
    b_out[...] = jnp.dot(xb, ws[...], preferred_element_type=jnp.float32) + bs[...]


def _stage1(x, Wk, bk, Wq, bq, Wv, bv, Ws, bias):
    nblk = N // ROW_BLK
    row_spec = pl.BlockSpec((ROW_BLK, D), lambda i: (i, 0))
    w_spec = pl.BlockSpec((D, D), lambda i: (0, 0))
    b_spec = pl.BlockSpec((1, D), lambda i: (0, 0))
    out = jax.ShapeDtypeStruct((N, D), jnp.float32)
    return pl.pallas_call(
        _mm_body,
        grid=(nblk,),
        in_specs=[row_spec, w_spec, b_spec, w_spec, b_spec, w_spec, b_spec,
                  w_spec, b_spec],
        out_specs=[row_spec, row_spec, row_spec, row_spec],
        out_shape=[out, out, out, out],
    )(x, Wk, bk.reshape(1, D), Wq, bq.reshape(1, D), Wv, bv.reshape(1, D),
      Ws, bias.reshape(1, D))


def _sc_body(src_hbm, dst_hbm, k_hbm, q_hbm, v_hbm, base_hbm, zero_hbm,
             p0_hbm, p1_hbm,
             dsti, srci, kd, qs, vs, agg, sem0, sem1, sem2):
    cid = lax.axis_index("c")
    sid = lax.axis_index("s")
    wid = cid * NS + sid

    @pl.when(sid == 0)
    def _():
        @pl.when(cid == 0)
        def _():
            pltpu.sync_copy(base_hbm, agg)

        @pl.when(cid == 1)
        def _():
            pltpu.sync_copy(zero_hbm, agg)

    plsc.subcore_barrier()

    base_edge = wid * EPT

    def chunk_body(t, carry):
        off = pl.multiple_of(base_edge + t * C, 8)
        pltpu.sync_copy(dst_hbm.at[pl.ds(off, C)], dsti)
        pltpu.sync_copy(src_hbm.at[pl.ds(off, C)], srci)
        cp_k = pltpu.async_copy(k_hbm.at[dsti], kd, sem0)
        cp_q = pltpu.async_copy(q_hbm.at[srci], qs, sem1)
        cp_v = pltpu.async_copy(v_hbm.at[srci], vs, sem2)
        cp_k.wait()
        cp_q.wait()
        cp_v.wait()

        def row_body(j, rcarry):
            for i in range(D // LANES):
                sl = pl.ds(i * LANES, LANES)
                z = kd[j, sl] + qs[j, sl]
                g = 1.0 / (1.0 + jnp.exp(-z))
                vs[j, sl] = g * vs[j, sl]
            return rcarry

        lax.fori_loop(0, C, row_body, 0)
        pltpu.sync_copy(vs, agg.at[dsti], add=True)
        return carry

    lax.fori_loop(0, NCHUNK, chunk_body, 0)

    plsc.subcore_barrier()

    rows_pt = N // NS
    roff = sid * rows_pt

    @pl.when(cid == 0)
    def _():
        pltpu.sync_copy(agg.at[pl.ds(roff, rows_pt)],
                        p0_hbm.at[pl.ds(roff, rows_pt)])

    @pl.when(cid == 1)
    def _():
        pltpu.sync_copy(agg.at[pl.ds(roff, rows_pt)],
                        p1_hbm.at[pl.ds(roff, rows_pt)])


_sc_edges = functools.partial(
    pl.kernel,
    out_type=[jax.ShapeDtypeStruct((N, D), jnp.float32)] * 2,
    mesh=plsc.VectorSubcoreMesh(core_axis_name="c", subcore_axis_name="s",
                                num_cores=NC, num_subcores=NS),
    scratch_types=[
        pltpu.VMEM((C,), jnp.int32),
        pltpu.VMEM((C,), jnp.int32),
        pltpu.VMEM((C, D), jnp.float32),
        pltpu.VMEM((C, D), jnp.float32),
        pltpu.VMEM((C, D), jnp.float32),
        pltpu.VMEM_SHARED((N, D), jnp.float32),
        pltpu.SemaphoreType.DMA,
        pltpu.SemaphoreType.DMA,
        pltpu.SemaphoreType.DMA,
    ],
)(_sc_body)


def _add_body(a_ref, b_ref, o_ref):
    o_ref[...] = a_ref[...] + b_ref[...]


def _stage3(p0, p1):
    row_spec = pl.BlockSpec((ROW_BLK, D), lambda i: (i, 0))
    return pl.pallas_call(
        _add_body,
        grid=(N // ROW_BLK,),
        in_specs=[row_spec, row_spec],
        out_specs=row_spec,
        out_shape=jax.ShapeDtypeStruct((N, D), jnp.float32),
    )(p0, p1)


def kernel(x, edge_index, Wk, bk, Wq, bq, Wv, bv, Ws, bias):
    src = edge_index[0]
    dst = edge_index[1]
    k, q, v, base = _stage1(x, Wk, bk, Wq, bq, Wv, bv, Ws, bias)
    zeros = jnp.zeros((N, D), dtype=jnp.float32)
    p0, p1 = _sc_edges(src, dst, k, q, v, base, zeros)
    return _stage3(p0, p1)


# speedup vs baseline: 5.4631x; 5.4631x over previous
import functools

import jax
import jax.numpy as jnp
from jax import lax
from jax.experimental import pallas as pl
from jax.experimental.pallas import tpu as pltpu
from jax.experimental.pallas import tpu_sc as plsc

N = 10000
E = 320000
D = 128

NC = 2
NS = 16
LANES = 16
C = 80
EPT = E // (NC * NS)
NCHUNK = EPT // C
ROW_BLK = 1000


def _mm_body(x_ref, wk, bk, wq, bq, wv, bv, ws, bs, k_out, q_out, v_out, b_out):
    xb = x_ref[...]
    k_out[...] = jnp.dot(xb, wk[...], preferred_element_type=jnp.float32) + bk[...]
    q_out[...] = jnp.dot(xb, wq[...], preferred_element_type=jnp.float32) + bq[...]
    v_out[...] = jnp.dot(xb, wv[...], preferred_element_type=jnp.float32) + bv[...]
    b_out[...] = jnp.dot(xb, ws[...], preferred_element_type=jnp.float32) + bs[...]


def _stage1(x, Wk, bk, Wq, bq, Wv, bv, Ws, bias):
    nblk = N // ROW_BLK
    row_spec = pl.BlockSpec((ROW_BLK, D), lambda i: (i, 0))
    w_spec = pl.BlockSpec((D, D), lambda i: (0, 0))
    b_spec = pl.BlockSpec((1, D), lambda i: (0, 0))
    out = jax.ShapeDtypeStruct((N, D), jnp.float32)
    return pl.pallas_call(
        _mm_body,
        grid=(nblk,),
        in_specs=[row_spec, w_spec, b_spec, w_spec, b_spec, w_spec, b_spec,
                  w_spec, b_spec],
        out_specs=[row_spec, row_spec, row_spec, row_spec],
        out_shape=[out, out, out, out],
    )(x, Wk, bk.reshape(1, D), Wq, bq.reshape(1, D), Wv, bv.reshape(1, D),
      Ws, bias.reshape(1, D))


def _sc_body(src_hbm, dst_hbm, k_hbm, q_hbm, v_hbm, base_hbm, zero_hbm,
             p0_hbm, p1_hbm,
             dsti, srci, kd, qs, vs, agg, sem0, sem1, sem2):
    cid = lax.axis_index("c")
    sid = lax.axis_index("s")
    wid = cid * NS + sid

    @pl.when(sid == 0)
    def _():
        @pl.when(cid == 0)
        def _():
            pltpu.sync_copy(base_hbm, agg)

        @pl.when(cid == 1)
        def _():
            pltpu.sync_copy(zero_hbm, agg)

    plsc.subcore_barrier()

    base_edge = wid * EPT

    def chunk_body(t, carry):
        off = pl.multiple_of(base_edge + t * C, 8)
        pltpu.sync_copy(dst_hbm.at[pl.ds(off, C)], dsti)
        pltpu.sync_copy(src_hbm.at[pl.ds(off, C)], srci)
        cp_k = pltpu.async_copy(k_hbm.at[dsti], kd, sem0)
        cp_q = pltpu.async_copy(q_hbm.at[srci], qs, sem1)
        cp_v = pltpu.async_copy(v_hbm.at[srci], vs, sem2)
        cp_k.wait()
        cp_q.wait()
        cp_v.wait()

        def row_body(j, rcarry):
            for i in range(D // LANES):
                sl = pl.ds(i * LANES, LANES)
                z = kd[j, sl] + qs[j, sl]
                g = 1.0 / (1.0 + jnp.exp(-z))
                vs[j, sl] = g * vs[j, sl]
            return rcarry

        lax.fori_loop(0, C, row_body, 0)
        pltpu.sync_copy(vs, agg.at[dsti], add=True)
        return carry

    lax.fori_loop(0, NCHUNK, chunk_body, 0)

    plsc.subcore_barrier()

    rows_pt = 624
    tail = N - (NS - 1) * rows_pt
    roff = pl.multiple_of(sid * rows_pt, 8)

    @pl.when(cid == 0)
    def _():
        @pl.when(sid < NS - 1)
        def _():
            pltpu.sync_copy(agg.at[pl.ds(roff, rows_pt)],
                            p0_hbm.at[pl.ds(roff, rows_pt)])

        @pl.when(sid == NS - 1)
        def _():
            pltpu.sync_copy(agg.at[pl.ds((NS - 1) * rows_pt, tail)],
                            p0_hbm.at[pl.ds((NS - 1) * rows_pt, tail)])

    @pl.when(cid == 1)
    def _():
        @pl.when(sid < NS - 1)
        def _():
            pltpu.sync_copy(agg.at[pl.ds(roff, rows_pt)],
                            p1_hbm.at[pl.ds(roff, rows_pt)])

        @pl.when(sid == NS - 1)
        def _():
            pltpu.sync_copy(agg.at[pl.ds((NS - 1) * rows_pt, tail)],
                            p1_hbm.at[pl.ds((NS - 1) * rows_pt, tail)])


@functools.lru_cache(maxsize=1)
def _sc_edges():
    return pl.kernel(
        _sc_body,
        out_type=[jax.ShapeDtypeStruct((N, D), jnp.float32)] * 2,
        mesh=plsc.VectorSubcoreMesh(core_axis_name="c", subcore_axis_name="s",
                                    num_cores=NC, num_subcores=NS),
        scratch_types=[
            pltpu.VMEM((C,), jnp.int32),
            pltpu.VMEM((C,), jnp.int32),
            pltpu.VMEM((C, D), jnp.float32),
            pltpu.VMEM((C, D), jnp.float32),
            pltpu.VMEM((C, D), jnp.float32),
            pltpu.VMEM_SHARED((N, D), jnp.float32),
            pltpu.SemaphoreType.DMA,
            pltpu.SemaphoreType.DMA,
            pltpu.SemaphoreType.DMA,
        ],
    )


def _add_body(a_ref, b_ref, o_ref):
    o_ref[...] = a_ref[...] + b_ref[...]


def _stage3(p0, p1):
    row_spec = pl.BlockSpec((ROW_BLK, D), lambda i: (i, 0))
    return pl.pallas_call(
        _add_body,
        grid=(N // ROW_BLK,),
        in_specs=[row_spec, row_spec],
        out_specs=row_spec,
        out_shape=jax.ShapeDtypeStruct((N, D), jnp.float32),
    )(p0, p1)


def kernel(x, edge_index, Wk, bk, Wq, bq, Wv, bv, Ws, bias):
    src = edge_index[0]
    dst = edge_index[1]
    k, q, v, base = _stage1(x, Wk, bk, Wq, bq, Wv, bv, Ws, bias)
    zeros = jnp.zeros((N, D), dtype=jnp.float32)
    p0, p1 = _sc_edges()(src, dst, k, q, v, base, zeros)
    return _stage3(p0, p1)


# speedup vs baseline: 7.8023x; 1.4282x over previous
import functools

import jax
import jax.numpy as jnp
from jax import lax
from jax.experimental import pallas as pl
from jax.experimental.pallas import tpu as pltpu
from jax.experimental.pallas import tpu_sc as plsc

N = 10000
E = 320000
D = 128

NC = 2
NS = 16
LANES = 16
C = 40
EPT = E // (NC * NS)
NCHUNK = EPT // C
ROW_BLK = 1000


def _mm_body(x_ref, wk, bk, wq, bq, wv, bv, ws, bs, k_out, q_out, v_out, b_out):
    xb = x_ref[...]
    k_out[...] = jnp.dot(xb, wk[...], preferred_element_type=jnp.float32) + bk[...]
    q_out[...] = jnp.dot(xb, wq[...], preferred_element_type=jnp.float32) + bq[...]
    v_out[...] = jnp.dot(xb, wv[...], preferred_element_type=jnp.float32) + bv[...]
    b_out[...] = jnp.dot(xb, ws[...], preferred_element_type=jnp.float32) + bs[...]


def _stage1(x, Wk, bk, Wq, bq, Wv, bv, Ws, bias):
    nblk = N // ROW_BLK
    row_spec = pl.BlockSpec((ROW_BLK, D), lambda i: (i, 0))
    w_spec = pl.BlockSpec((D, D), lambda i: (0, 0))
    b_spec = pl.BlockSpec((1, D), lambda i: (0, 0))
    out = jax.ShapeDtypeStruct((N, D), jnp.float32)
    return pl.pallas_call(
        _mm_body,
        grid=(nblk,),
        in_specs=[row_spec, w_spec, b_spec, w_spec, b_spec, w_spec, b_spec,
                  w_spec, b_spec],
        out_specs=[row_spec, row_spec, row_spec, row_spec],
        out_shape=[out, out, out, out],
    )(x, Wk, bk.reshape(1, D), Wq, bq.reshape(1, D), Wv, bv.reshape(1, D),
      Ws, bias.reshape(1, D))


def _sc_body(src_hbm, dst_hbm, k_hbm, q_hbm, v_hbm, base_hbm, zero_hbm,
             p0_hbm, p1_hbm,
             di0, si0, kd0, qs0, vs0, di1, si1, kd1, qs1, vs1, agg,
             semi0, semk0, semq0, semv0, semi1, semk1, semq1, semv1):
    cid = lax.axis_index("c")
    sid = lax.axis_index("s")
    wid = cid * NS + sid

    @pl.when(sid == 0)
    def _():
        @pl.when(cid == 0)
        def _():
            pltpu.sync_copy(base_hbm, agg)

        @pl.when(cid == 1)
        def _():
            pltpu.sync_copy(zero_hbm, agg)

    plsc.subcore_barrier()

    bufs = ((di0, si0, kd0, qs0, vs0, semi0, semk0, semq0, semv0),
            (di1, si1, kd1, qs1, vs1, semi1, semk1, semq1, semv1))

    def idx_start(b, t):
        di, si, kd, qs, vs, smi, sk, sq, sv = bufs[b]
        pltpu.async_copy(dst_hbm.at[wid, pl.ds(t, 1)], di, smi)
        pltpu.async_copy(src_hbm.at[wid, pl.ds(t, 1)], si, smi)

    def idx_wait(b, t):
        di, si, kd, qs, vs, smi, sk, sq, sv = bufs[b]
        pltpu.make_async_copy(dst_hbm.at[wid, pl.ds(t, 1)], di, smi).wait()
        pltpu.make_async_copy(src_hbm.at[wid, pl.ds(t, 1)], si, smi).wait()

    def start_g(b, t):
        di, si, kd, qs, vs, smi, sk, sq, sv = bufs[b]
        idx_wait(b, t)
        pltpu.async_copy(k_hbm.at[di.at[0]], kd, sk)
        pltpu.async_copy(q_hbm.at[si.at[0]], qs, sq)
        pltpu.async_copy(v_hbm.at[si.at[0]], vs, sv)

    def finish(b, t):
        di, si, kd, qs, vs, smi, sk, sq, sv = bufs[b]
        pltpu.make_async_copy(k_hbm.at[di.at[0]], kd, sk).wait()
        pltpu.make_async_copy(q_hbm.at[si.at[0]], qs, sq).wait()
        pltpu.make_async_copy(v_hbm.at[si.at[0]], vs, sv).wait()

        def row_body(j, rcarry):
            for i in range(D // LANES):
                sl = pl.ds(i * LANES, LANES)
                z = kd[j, sl] + qs[j, sl]
                g = 1.0 / (1.0 + jnp.exp(-z))
                vs[j, sl] = g * vs[j, sl]
            return rcarry

        lax.fori_loop(0, C, row_body, 0)
        pltpu.sync_copy(vs, agg.at[di.at[0]], add=True)

        @pl.when(t + 2 < NCHUNK)
        def _():
            idx_start(b, t + 2)

    idx_start(0, 0)
    idx_start(1, 1)
    start_g(0, 0)

    def pair_body(it, carry):
        c0 = it * 2
        c1 = c0 + 1
        start_g(1, c1)
        finish(0, c0)

        @pl.when(c1 + 1 < NCHUNK)
        def _():
            start_g(0, c1 + 1)

        finish(1, c1)
        return carry

    lax.fori_loop(0, NCHUNK // 2, pair_body, 0)

    plsc.subcore_barrier()

    rows_pt = 624
    tail = N - (NS - 1) * rows_pt
    roff = pl.multiple_of(sid * rows_pt, 8)

    @pl.when(cid == 0)
    def _():
        @pl.when(sid < NS - 1)
        def _():
            pltpu.sync_copy(agg.at[pl.ds(roff, rows_pt)],
                            p0_hbm.at[pl.ds(roff, rows_pt)])

        @pl.when(sid == NS - 1)
        def _():
            pltpu.sync_copy(agg.at[pl.ds((NS - 1) * rows_pt, tail)],
                            p0_hbm.at[pl.ds((NS - 1) * rows_pt, tail)])

    @pl.when(cid == 1)
    def _():
        @pl.when(sid < NS - 1)
        def _():
            pltpu.sync_copy(agg.at[pl.ds(roff, rows_pt)],
                            p1_hbm.at[pl.ds(roff, rows_pt)])

        @pl.when(sid == NS - 1)
        def _():
            pltpu.sync_copy(agg.at[pl.ds((NS - 1) * rows_pt, tail)],
                            p1_hbm.at[pl.ds((NS - 1) * rows_pt, tail)])


@functools.lru_cache(maxsize=1)
def _sc_edges():
    return pl.kernel(
        _sc_body,
        out_type=[jax.ShapeDtypeStruct((N, D), jnp.float32)] * 2,
        mesh=plsc.VectorSubcoreMesh(core_axis_name="c", subcore_axis_name="s",
                                    num_cores=NC, num_subcores=NS),
        scratch_types=[
            pltpu.VMEM((1, C), jnp.int32),
            pltpu.VMEM((1, C), jnp.int32),
            pltpu.VMEM((C, D), jnp.float32),
            pltpu.VMEM((C, D), jnp.float32),
            pltpu.VMEM((C, D), jnp.float32),
            pltpu.VMEM((1, C), jnp.int32),
            pltpu.VMEM((1, C), jnp.int32),
            pltpu.VMEM((C, D), jnp.float32),
            pltpu.VMEM((C, D), jnp.float32),
            pltpu.VMEM((C, D), jnp.float32),
            pltpu.VMEM_SHARED((N, D), jnp.float32),
            pltpu.SemaphoreType.DMA,
            pltpu.SemaphoreType.DMA,
            pltpu.SemaphoreType.DMA,
            pltpu.SemaphoreType.DMA,
            pltpu.SemaphoreType.DMA,
            pltpu.SemaphoreType.DMA,
            pltpu.SemaphoreType.DMA,
            pltpu.SemaphoreType.DMA,
        ],
    )


def _add_body(a_ref, b_ref, o_ref):
    o_ref[...] = a_ref[...] + b_ref[...]


def _stage3(p0, p1):
    row_spec = pl.BlockSpec((ROW_BLK, D), lambda i: (i, 0))
    return pl.pallas_call(
        _add_body,
        grid=(N // ROW_BLK,),
        in_specs=[row_spec, row_spec],
        out_specs=row_spec,
        out_shape=jax.ShapeDtypeStruct((N, D), jnp.float32),
    )(p0, p1)


def kernel(x, edge_index, Wk, bk, Wq, bq, Wv, bv, Ws, bias):
    src = edge_index[0].reshape(NC * NS, NCHUNK, C)
    dst = edge_index[1].reshape(NC * NS, NCHUNK, C)
    k, q, v, base = _stage1(x, Wk, bk, Wq, bq, Wv, bv, Ws, bias)
    zeros = jnp.zeros((N, D), dtype=jnp.float32)
    p0, p1 = _sc_edges()(src, dst, k, q, v, base, zeros)
    return _stage3(p0, p1)
